# baseline (speedup 1.0000x reference)
import jax
import jax.numpy as jnp
import numpy as np
from jax import lax
from jax.experimental import pallas as pl
from jax.experimental.pallas import tpu as pltpu
from jax.experimental.pallas import tpu_sc as plsc

NSEG = 256
TOTAL = (NSEG * (NSEG - 1)) // 2
NW = 32
SEG_PER_W = NSEG // NW
L = 16
NEG = np.float32(-3e38)
N_BISECT = 24
WSLOT = 272

_NMAX = [(NW - 1) + NW * j for j in range(SEG_PER_W)]
_LEN = [((15 + nm + 7) // 8) * 8 for nm in _NMAX]
_C = [(15 + nm + 15) // 16 for nm in _NMAX]
_ZOFF = np.concatenate([[0], np.cumsum([16 * c for c in _C])]).astype(int)
_ZTOT = int(_ZOFF[-1])
_R = [(nm + 127) // 128 for nm in _NMAX]
_ROFF = np.concatenate([[0], np.cumsum(_R)]).astype(int)
_RTOT = int(_ROFF[-1])


def _tec_body(x_hbm, gsl_hbm, out_hbm, win, zbuf, idxb, valb, gslv, dsem):
    wid = lax.axis_index("s") * 2 + lax.axis_index("c")
    lanes = lax.iota(jnp.int32, L)

    def bcast(v):
        return lax.broadcast_in_dim(v, (L,), ())

    ns, offs, shifts, copies = [], [], [], []
    copies.append(pltpu.async_copy(gsl_hbm, gslv, dsem))
    for j in range(SEG_PER_W):
        n = wid + NW * j
        off = (n * (n - 1)) // 2
        start8 = jnp.minimum((off // 8) * 8, TOTAL - _LEN[j])
        start8 = pl.multiple_of(start8, 8)
        ns.append(n)
        offs.append(off)
        shifts.append(off - start8)
        copies.append(pltpu.async_copy(
            x_hbm.at[pl.ds(start8, _LEN[j])],
            win.at[pl.ds(j * WSLOT, _LEN[j])], dsem))
    for cp in copies:
        cp.wait()

    lo, hi = [], []
    for j in range(SEG_PER_W):
        n, shift = ns[j], shifts[j]
        m = None
        for c in range(_C[j]):
            v = win[pl.ds(j * WSLOT + shift + 16 * c, 16)]
            pos = lanes + 16 * c
            z = jnp.where(pos < n, v, NEG)
            zbuf[pl.ds(int(_ZOFF[j]) + 16 * c, 16)] = z
            m = z if m is None else jnp.maximum(m, z)
        zmaxv = bcast(jnp.max(m))
        n_fv = jnp.maximum(bcast(n).astype(jnp.float32), 1.0)
        lo.append(zmaxv - 1.0)
        hi.append(zmaxv - 1.0 / n_fv)

    def bis(_, carry):
        los, his = carry
        nlos, nhis = [], []
        for j in range(SEG_PER_W):
            mid = 0.5 * (los[j] + his[j])
            acc = None
            for c in range(_C[j]):
                z = zbuf[pl.ds(int(_ZOFF[j]) + 16 * c, 16)]
                r = jnp.maximum(z - mid, 0.0)
                acc = r if acc is None else acc + r
            big = bcast(jnp.sum(acc)) > 1.0
            nlos.append(jnp.where(big, mid, los[j]))
            nhis.append(jnp.where(big, his[j], mid))
        return (tuple(nlos), tuple(nhis))

    lo, hi = lax.fori_loop(0, N_BISECT, bis, (tuple(lo), tuple(hi)))

    for j in range(SEG_PER_W):
        tau0 = 0.5 * (lo[j] + hi[j])
        cnt = None
        ssum = None
        for c in range(_C[j]):
            z = zbuf[pl.ds(int(_ZOFF[j]) + 16 * c, 16)]
            msk = z > tau0
            c1 = jnp.where(msk, 1.0, 0.0)
            s1 = jnp.where(msk, z, 0.0)
            cnt = c1 if cnt is None else cnt + c1
            ssum = s1 if ssum is None else ssum + s1
        tau = ((bcast(jnp.sum(ssum)) - 1.0) /
               jnp.maximum(bcast(jnp.sum(cnt)), 1.0))
        n, off = ns[j], offs[j]
        multv = plsc.load_gather(
            gslv, [jnp.zeros((L,), jnp.int32) + n]).astype(jnp.float32)
        nm1 = jnp.maximum(n - 1, 0)
        for c in range(8 * _R[j]):
            pos = lanes + 16 * c
            qpos = jnp.minimum(pos, nm1)
            zq = plsc.load_gather(zbuf, [int(_ZOFF[j]) + qpos])
            row = int(_ROFF[j]) + (c // 8)
            col = 16 * (c % 8)
            idxb[row, pl.ds(col, 16)] = off + qpos
            valb[row, pl.ds(col, 16)] = jnp.maximum(zq - tau, 0.0) * multv

    out_copies = []
    for j in range(1, SEG_PER_W):
        for r in range(_R[j]):
            row = int(_ROFF[j]) + r
            out_copies.append(pltpu.async_copy(
                valb.at[row], out_hbm.at[idxb.at[row]], dsem))

    @pl.when(wid > 0)
    def _():
        pltpu.async_copy(
            valb.at[int(_ROFF[0])], out_hbm.at[idxb.at[int(_ROFF[0])]],
            dsem).wait()

    for cp in out_copies:
        cp.wait()


def kernel(x, graph_size_list):
    mesh = plsc.VectorSubcoreMesh(core_axis_name="c", subcore_axis_name="s")
    launch = pl.kernel(
        _tec_body,
        mesh=mesh,
        compiler_params=pltpu.CompilerParams(needs_layout_passes=False),
        out_type=jax.ShapeDtypeStruct((TOTAL,), jnp.float32),
        scratch_types=[
            pltpu.VMEM((SEG_PER_W * WSLOT,), jnp.float32),
            pltpu.VMEM((_ZTOT,), jnp.float32),
            pltpu.VMEM((_RTOT, 128), jnp.int32),
            pltpu.VMEM((_RTOT, 128), jnp.float32),
            pltpu.VMEM((256,), jnp.int32),
            pltpu.SemaphoreType.DMA,
        ],
    )
    return launch(x, graph_size_list)


# speedup vs baseline: 13.6104x; 13.6104x over previous
import jax
import jax.numpy as jnp
import numpy as np
from jax import lax
from jax.experimental import pallas as pl
from jax.experimental.pallas import tpu as pltpu
from jax.experimental.pallas import tpu_sc as plsc

NSEG = 256
TOTAL = (NSEG * (NSEG - 1)) // 2
L = 16
NEG = np.float32(-3e38)
N_BISECT = 24
WSLOT = 272
BBASE = 8192
SHIM = BBASE + 14304

_NMAX = [79, 95, 111, 127, 207, 223, 239, 255]
_LEN = [((15 + nm + 7) // 8) * 8 for nm in _NMAX]
_C = [(nm + 15) // 16 for nm in _NMAX]
_ZOFF = np.concatenate([[0], np.cumsum([16 * c for c in _C])]).astype(int)
_ZTOT = int(_ZOFF[-1])
_R = [(nm + 127) // 128 for nm in _NMAX]
_ROFF = np.concatenate([[0], np.cumsum(_R)]).astype(int)
_RTOT = int(_ROFF[-1])

_COPY0 = [(128, 1888, 0, 0), (896, 13408, 18336, BBASE)]
_COPY1 = [(384, 5728, 2016, 0), (640, 9568, 8128, BBASE)]


def _tec_body(x_hbm, gsl_hbm, out_hbm, win, zbuf, idxb, valb, gslv, shim,
              dsem):
    core = lax.axis_index("c")
    sub = lax.axis_index("s")
    lanes = lax.iota(jnp.int32, L)
    a0 = jnp.where(core == 0, 0, 64)
    b0 = jnp.where(core == 0, 192, 128)
    ca = jnp.where(core == 0, 0, 2016)
    cb = jnp.where(core == 0, 18336, 8128)

    def bcast(v):
        return lax.broadcast_in_dim(v, (L,), ())

    ns, lbases, shifts, copies = [], [], [], []
    copies.append(pltpu.async_copy(gsl_hbm, gslv, dsem))
    for j in range(8):
        if j < 4:
            n = a0 + sub + 16 * j
        else:
            n = b0 + sub + 16 * (j - 4)
        off = (n * (n - 1)) // 2
        start8 = jnp.minimum((off // 8) * 8, TOTAL - _LEN[j])
        start8 = pl.multiple_of(start8, 8)
        ns.append(n)
        lbases.append(off - ca if j < 4 else BBASE + (off - cb))
        shifts.append(off - start8)
        copies.append(pltpu.async_copy(
            x_hbm.at[pl.ds(start8, _LEN[j])],
            win.at[pl.ds(j * WSLOT, _LEN[j])], dsem))
    for cp in copies:
        cp.wait()

    lo, hi = [], []
    for j in range(8):
        n, shift = ns[j], shifts[j]
        m = None
        for c in range(_C[j]):
            v = win[pl.ds(j * WSLOT + shift + 16 * c, 16)]
            pos = lanes + 16 * c
            z = jnp.where(pos < n, v, NEG)
            zbuf[pl.ds(int(_ZOFF[j]) + 16 * c, 16)] = z
            m = z if m is None else jnp.maximum(m, z)
        zmaxv = bcast(jnp.max(m))
        n_fv = jnp.maximum(bcast(n).astype(jnp.float32), 1.0)
        lo.append(zmaxv - 1.0)
        hi.append(zmaxv - 1.0 / n_fv)

    def bis(_, carry):
        los, his = carry
        nlos, nhis = [], []
        for j in range(8):
            mid = 0.5 * (los[j] + his[j])
            acc = None
            for c in range(_C[j]):
                z = zbuf[pl.ds(int(_ZOFF[j]) + 16 * c, 16)]
                r = jnp.maximum(z - mid, 0.0)
                acc = r if acc is None else acc + r
            big = bcast(jnp.sum(acc)) > 1.0
            nlos.append(jnp.where(big, mid, los[j]))
            nhis.append(jnp.where(big, his[j], mid))
        return (tuple(nlos), tuple(nhis))

    lo, hi = lax.fori_loop(0, N_BISECT, bis, (tuple(lo), tuple(hi)))

    for j in range(8):
        tau0 = 0.5 * (lo[j] + hi[j])
        cnt = None
        ssum = None
        for c in range(_C[j]):
            z = zbuf[pl.ds(int(_ZOFF[j]) + 16 * c, 16)]
            msk = z > tau0
            c1 = jnp.where(msk, 1.0, 0.0)
            s1 = jnp.where(msk, z, 0.0)
            cnt = c1 if cnt is None else cnt + c1
            ssum = s1 if ssum is None else ssum + s1
        tau = ((bcast(jnp.sum(ssum)) - 1.0) /
               jnp.maximum(bcast(jnp.sum(cnt)), 1.0))
        n = ns[j]
        multv = plsc.load_gather(
            gslv, [jnp.zeros((L,), jnp.int32) + n]).astype(jnp.float32)
        nm1 = jnp.maximum(n - 1, 0)
        for c in range(8 * _R[j]):
            pos = lanes + 16 * c
            qpos = jnp.minimum(pos, nm1)
            zq = plsc.load_gather(zbuf, [int(_ZOFF[j]) + qpos])
            row = int(_ROFF[j]) + (c // 8)
            col = 16 * (c % 8)
            idxb[row, pl.ds(col, 16)] = lbases[j] + qpos
            valb[row, pl.ds(col, 16)] = jnp.maximum(zq - tau, 0.0) * multv

    out_copies = []
    for j in range(1, 8):
        for r in range(_R[j]):
            row = int(_ROFF[j]) + r
            out_copies.append(pltpu.async_copy(
                valb.at[row], shim.at[idxb.at[row]], dsem))

    @pl.when(ns[0] > 0)
    def _():
        pltpu.async_copy(
            valb.at[int(_ROFF[0])], shim.at[idxb.at[int(_ROFF[0])]],
            dsem).wait()

    for cp in out_copies:
        cp.wait()

    plsc.subcore_barrier()

    @pl.when(core == 0)
    def _():
        for k, (ln, mx, hb, lb) in enumerate(_COPY0):
            o = pl.multiple_of(jnp.minimum(ln * sub, mx), 8)
            pltpu.sync_copy(shim.at[pl.ds(lb + o, ln)],
                            win.at[pl.ds(1024 * k, ln)])
            pltpu.sync_copy(win.at[pl.ds(1024 * k, ln)],
                            out_hbm.at[pl.ds(hb + o, ln)])

    @pl.when(core == 1)
    def _():
        for k, (ln, mx, hb, lb) in enumerate(_COPY1):
            o = pl.multiple_of(jnp.minimum(ln * sub, mx), 8)
            pltpu.sync_copy(shim.at[pl.ds(lb + o, ln)],
                            win.at[pl.ds(1024 * k, ln)])
            pltpu.sync_copy(win.at[pl.ds(1024 * k, ln)],
                            out_hbm.at[pl.ds(hb + o, ln)])


def kernel(x, graph_size_list):
    mesh = plsc.VectorSubcoreMesh(core_axis_name="c", subcore_axis_name="s")
    launch = pl.kernel(
        _tec_body,
        mesh=mesh,
        compiler_params=pltpu.CompilerParams(needs_layout_passes=False),
        out_type=jax.ShapeDtypeStruct((TOTAL,), jnp.float32),
        scratch_types=[
            pltpu.VMEM((8 * WSLOT + 32,), jnp.float32),
            pltpu.VMEM((_ZTOT,), jnp.float32),
            pltpu.VMEM((_RTOT, 128), jnp.int32),
            pltpu.VMEM((_RTOT, 128), jnp.float32),
            pltpu.VMEM((256,), jnp.int32),
            pltpu.VMEM_SHARED((SHIM,), jnp.float32),
            pltpu.SemaphoreType.DMA,
        ],
    )
    return launch(x, graph_size_list)


# speedup vs baseline: 15.1640x; 1.1142x over previous
import jax
import jax.numpy as jnp
import numpy as np
from jax import lax
from jax.experimental import pallas as pl
from jax.experimental.pallas import tpu as pltpu
from jax.experimental.pallas import tpu_sc as plsc

NSEG = 256
TOTAL = (NSEG * (NSEG - 1)) // 2
L = 16
NEG = np.float32(-3e38)
N_BISECT = 14
N_POLISH = 2
WSLOT = 272
BBASE = 8192
SHIM = BBASE + 14304

_NMAX = [79, 95, 111, 127, 207, 223, 239, 255]
_LEN = [((15 + nm + 7) // 8) * 8 for nm in _NMAX]
_C = [(nm + 15) // 16 for nm in _NMAX]
_ZOFF = np.concatenate([[0], np.cumsum([16 * c for c in _C])]).astype(int)
_ZTOT = int(_ZOFF[-1])
_R = [(nm + 127) // 128 for nm in _NMAX]
_ROFF = np.concatenate([[0], np.cumsum(_R)]).astype(int)
_RTOT = int(_ROFF[-1])

_COPY0 = [(128, 1888, 0, 0), (896, 13408, 18336, BBASE)]
_COPY1 = [(384, 5728, 2016, 0), (640, 9568, 8128, BBASE)]


def _tec_body(x_hbm, gsl_hbm, out_hbm, win, zbuf, idxb, valb, gslv, shim,
              dsem):
    core = lax.axis_index("c")
    sub = lax.axis_index("s")
    lanes = lax.iota(jnp.int32, L)
    a0 = jnp.where(core == 0, 0, 64)
    b0 = jnp.where(core == 0, 192, 128)
    ca = jnp.where(core == 0, 0, 2016)
    cb = jnp.where(core == 0, 18336, 8128)

    def bcast(v):
        return lax.broadcast_in_dim(v, (L,), ())

    ns, lbases, shifts, copies = [], [], [], []
    copies.append(pltpu.async_copy(gsl_hbm, gslv, dsem))
    for j in range(8):
        if j < 4:
            n = a0 + sub + 16 * j
        else:
            n = b0 + sub + 16 * (j - 4)
        off = (n * (n - 1)) // 2
        start8 = jnp.minimum((off // 8) * 8, TOTAL - _LEN[j])
        start8 = pl.multiple_of(start8, 8)
        ns.append(n)
        lbases.append(off - ca if j < 4 else BBASE + (off - cb))
        shifts.append(off - start8)
        copies.append(pltpu.async_copy(
            x_hbm.at[pl.ds(start8, _LEN[j])],
            win.at[pl.ds(j * WSLOT, _LEN[j])], dsem))
    for cp in copies:
        cp.wait()

    lo, hi = [], []
    for j in range(8):
        n, shift = ns[j], shifts[j]
        m = None
        for c in range(_C[j]):
            v = win[pl.ds(j * WSLOT + shift + 16 * c, 16)]
            pos = lanes + 16 * c
            z = jnp.where(pos < n, v, NEG)
            zbuf[pl.ds(int(_ZOFF[j]) + 16 * c, 16)] = z
            m = z if m is None else jnp.maximum(m, z)
        zmaxv = bcast(jnp.max(m))
        n_fv = jnp.maximum(bcast(n).astype(jnp.float32), 1.0)
        lo.append(zmaxv - 1.0)
        hi.append(zmaxv - 1.0 / n_fv)

    def bis(_, carry):
        los, his = carry
        nlos, nhis = [], []
        for j in range(8):
            mid = 0.5 * (los[j] + his[j])
            acc = None
            for c in range(_C[j]):
                z = zbuf[pl.ds(int(_ZOFF[j]) + 16 * c, 16)]
                r = jnp.maximum(z - mid, 0.0)
                acc = r if acc is None else acc + r
            big = bcast(jnp.sum(acc)) > 1.0
            nlos.append(jnp.where(big, mid, los[j]))
            nhis.append(jnp.where(big, his[j], mid))
        return (tuple(nlos), tuple(nhis))

    lo, hi = lax.fori_loop(0, N_BISECT, bis, (tuple(lo), tuple(hi)))

    for j in range(8):
        tau = 0.5 * (lo[j] + hi[j])
        for _ in range(N_POLISH):
            cnt = None
            ssum = None
            for c in range(_C[j]):
                z = zbuf[pl.ds(int(_ZOFF[j]) + 16 * c, 16)]
                msk = z > tau
                c1 = jnp.where(msk, 1.0, 0.0)
                s1 = jnp.where(msk, z, 0.0)
                cnt = c1 if cnt is None else cnt + c1
                ssum = s1 if ssum is None else ssum + s1
            tau = ((bcast(jnp.sum(ssum)) - 1.0) /
                   jnp.maximum(bcast(jnp.sum(cnt)), 1.0))
        n = ns[j]
        multv = plsc.load_gather(
            gslv, [jnp.zeros((L,), jnp.int32) + n]).astype(jnp.float32)
        nm1 = jnp.maximum(n - 1, 0)
        vlast = plsc.load_gather(
            zbuf, [jnp.zeros((L,), jnp.int32) + (int(_ZOFF[j]) + nm1)])
        olast = jnp.maximum(vlast - tau, 0.0) * multv
        for c in range(8 * _R[j]):
            pos = lanes + 16 * c
            valid = pos < n
            row = int(_ROFF[j]) + (c // 8)
            col = 16 * (c % 8)
            if c < _C[j]:
                z = zbuf[pl.ds(int(_ZOFF[j]) + 16 * c, 16)]
                val = jnp.where(valid,
                                jnp.maximum(z - tau, 0.0) * multv, olast)
            else:
                val = olast
            idxb[row, pl.ds(col, 16)] = (
                lbases[j] + jnp.where(valid, pos, nm1))
            valb[row, pl.ds(col, 16)] = val

    out_copies = []
    for j in range(1, 8):
        for r in range(_R[j]):
            row = int(_ROFF[j]) + r
            out_copies.append(pltpu.async_copy(
                valb.at[row], shim.at[idxb.at[row]], dsem))

    @pl.when(ns[0] > 0)
    def _():
        pltpu.async_copy(
            valb.at[int(_ROFF[0])], shim.at[idxb.at[int(_ROFF[0])]],
            dsem).wait()

    for cp in out_copies:
        cp.wait()

    plsc.subcore_barrier()

    def slabs(table):
        offs_ = [pl.multiple_of(jnp.minimum(ln * sub, mx), 8)
                 for (ln, mx, _, _) in table]
        cps = [pltpu.async_copy(shim.at[pl.ds(lb + o, ln)],
                                win.at[pl.ds(1024 * k, ln)], dsem)
               for k, ((ln, mx, hb, lb), o) in enumerate(zip(table, offs_))]
        for cp in cps:
            cp.wait()
        cps = [pltpu.async_copy(win.at[pl.ds(1024 * k, ln)],
                                out_hbm.at[pl.ds(hb + o, ln)], dsem)
               for k, ((ln, mx, hb, lb), o) in enumerate(zip(table, offs_))]
        for cp in cps:
            cp.wait()

    @pl.when(core == 0)
    def _():
        slabs(_COPY0)

    @pl.when(core == 1)
    def _():
        slabs(_COPY1)


def kernel(x, graph_size_list):
    mesh = plsc.VectorSubcoreMesh(core_axis_name="c", subcore_axis_name="s")
    launch = pl.kernel(
        _tec_body,
        mesh=mesh,
        compiler_params=pltpu.CompilerParams(needs_layout_passes=False),
        out_type=jax.ShapeDtypeStruct((TOTAL,), jnp.float32),
        scratch_types=[
            pltpu.VMEM((8 * WSLOT + 32,), jnp.float32),
            pltpu.VMEM((_ZTOT,), jnp.float32),
            pltpu.VMEM((_RTOT, 128), jnp.int32),
            pltpu.VMEM((_RTOT, 128), jnp.float32),
            pltpu.VMEM((256,), jnp.int32),
            pltpu.VMEM_SHARED((SHIM,), jnp.float32),
            pltpu.SemaphoreType.DMA,
        ],
    )
    return launch(x, graph_size_list)
